# baseline (speedup 1.0000x reference)
import jax
import jax.numpy as jnp
from jax import lax
from jax.experimental import pallas as pl
from jax.experimental.pallas import tpu as pltpu
from jax.experimental.pallas import tpu_sc as plsc

N_CLS = 100
N_CTX = 16
CTX_DIM = 512
SEQ_LEN = 77
SUFFIX_LEN = SEQ_LEN - 1 - N_CTX
BATCH = 1024

NC, NS, L = 2, 16, 16
NW = NC * NS
BPW = BATCH // NW

TOK_PAD = 128
ROW_STRIDE = 80
NBUF = 3


def _sc_gather_body(label_hbm, tbl_hbm, tok_hbm, out_hbm, tokout_hbm,
                    idx_v, ridx_v, stage0, stage1, stage2, tokbuf,
                    gsems, ssems, tg_sem, ts_sem):
  wid = lax.axis_index("s") * NC + lax.axis_index("c")
  base = wid * BPW
  stages = (stage0, stage1, stage2)

  pltpu.sync_copy(label_hbm.at[pl.ds(base, BPW)], idx_v)

  iota = lax.iota(jnp.int32, L)

  for g in range(BPW // L):
    lab = idx_v[pl.ds(g * L, L)]
    pos = iota * ROW_STRIDE + g * L * ROW_STRIDE
    for k in range(ROW_STRIDE):
      plsc.store_scatter(ridx_v, [pos + k],
                         lab * SEQ_LEN + min(k, SEQ_LEN - 1))

  tokc = None
  for g in range(BPW // L):
    b0 = base + g * L
    if g > 0:
      tokc.wait()
    tg = pltpu.make_async_copy(tok_hbm.at[idx_v.at[pl.ds(g * L, L)]],
                               tokbuf, tg_sem)
    tg.start()
    tg.wait()
    tokc = pltpu.make_async_copy(tokbuf, tokout_hbm.at[pl.ds(b0, L)], ts_sem)
    tokc.start()

  scatters = {}
  gathers = {}

  def start_gather(i):
    r = i % NBUF
    h = pltpu.make_async_copy(
        tbl_hbm.at[ridx_v.at[pl.ds(i * ROW_STRIDE, ROW_STRIDE)]],
        stages[r], gsems[r])
    h.start()
    gathers[i] = (h,)

  start_gather(0)
  start_gather(1)
  for i in range(BPW):
    r = i % NBUF
    if i + 2 < BPW:
      if i + 2 >= NBUF:
        for sh in scatters.pop(i + 2 - NBUF):
          sh.wait()
      start_gather(i + 2)
    for h in gathers.pop(i):
      h.wait()
    sh0 = pltpu.make_async_copy(stages[r].at[pl.ds(0, 72)],
                                out_hbm.at[base + i, pl.ds(0, 72)], ssems[r])
    sh1 = pltpu.make_async_copy(stages[r].at[pl.ds(72, 5)],
                                out_hbm.at[base + i, pl.ds(72, 5)], ssems[r])
    sh0.start()
    sh1.start()
    scatters[i] = (sh0, sh1)

  for i in sorted(scatters):
    for sh in scatters[i]:
      sh.wait()
  tokc.wait()


@jax.jit
def _prompt_gather(label, ctx, token_prefix, token_suffix, tokenized_prompts):
  sfx_std = lax.optimization_barrier(
      token_suffix.reshape(N_CLS * SUFFIX_LEN, CTX_DIM)).reshape(
          N_CLS, SUFFIX_LEN, CTX_DIM)
  tbl = jnp.concatenate([token_prefix, ctx, sfx_std],
                        axis=1).reshape(N_CLS * SEQ_LEN, CTX_DIM)
  tok_r = jnp.pad(tokenized_prompts, ((0, 0), (0, TOK_PAD - SEQ_LEN)))

  mesh = plsc.VectorSubcoreMesh(core_axis_name="c", subcore_axis_name="s")
  run = pl.kernel(
      _sc_gather_body,
      out_type=(
          jax.ShapeDtypeStruct((BATCH, SEQ_LEN, CTX_DIM), jnp.float32),
          jax.ShapeDtypeStruct((BATCH, TOK_PAD), jnp.int32),
      ),
      mesh=mesh,
      scratch_types=[
          pltpu.VMEM((BPW,), jnp.int32),
          pltpu.VMEM((BPW * ROW_STRIDE,), jnp.int32),
          pltpu.VMEM((ROW_STRIDE, CTX_DIM), jnp.float32),
          pltpu.VMEM((ROW_STRIDE, CTX_DIM), jnp.float32),
          pltpu.VMEM((ROW_STRIDE, CTX_DIM), jnp.float32),
          pltpu.VMEM((L, TOK_PAD), jnp.int32),
          [pltpu.SemaphoreType.DMA] * NBUF,
          [pltpu.SemaphoreType.DMA] * NBUF,
          pltpu.SemaphoreType.DMA,
          pltpu.SemaphoreType.DMA,
      ],
      compiler_params=pltpu.CompilerParams(needs_layout_passes=False),
  )
  prompts, tok_padded = run(label, tbl, tok_r)
  return prompts, tok_padded[:, :SEQ_LEN]


def kernel(label, ctx, token_prefix, token_suffix, tokenized_prompts):
  return _prompt_gather(label, ctx, token_prefix, token_suffix,
                        tokenized_prompts)


# speedup vs baseline: 1.0751x; 1.0751x over previous
import jax
import jax.numpy as jnp
from jax import lax
from jax.experimental import pallas as pl
from jax.experimental.pallas import tpu as pltpu
from jax.experimental.pallas import tpu_sc as plsc

N_CLS = 100
N_CTX = 16
CTX_DIM = 512
SEQ_LEN = 77
SUFFIX_LEN = SEQ_LEN - 1 - N_CTX
BATCH = 1024

NC, NS, L = 2, 16, 16
NW = NC * NS
BPW = BATCH // NW

TOK_PAD = 128
ROW_STRIDE = 80
NBUF = 3


def _sc_gather_body(label_hbm, tbl_hbm, tok_hbm, out_hbm, tokout_hbm,
                    idx_v, ridx_v, stage0, stage1, stage2, tokbuf,
                    gsems, ssems, tg_sem, ts_sem):
  wid = lax.axis_index("s") * NC + lax.axis_index("c")
  base = wid * BPW
  stages = (stage0, stage1, stage2)

  pltpu.sync_copy(label_hbm.at[pl.ds(base, BPW)], idx_v)

  iota = lax.iota(jnp.int32, L)

  for g in range(BPW // L):
    lab = idx_v[pl.ds(g * L, L)]
    pos = iota * ROW_STRIDE + g * L * ROW_STRIDE
    for k in range(ROW_STRIDE):
      plsc.store_scatter(ridx_v, [pos + k],
                         lab * SEQ_LEN + min(k, SEQ_LEN - 1))

  tokc = None
  for g in range(BPW // L):
    b0 = base + g * L
    if g > 0:
      tokc.wait()
    tg = pltpu.make_async_copy(tok_hbm.at[idx_v.at[pl.ds(g * L, L)]],
                               tokbuf, tg_sem)
    tg.start()
    tg.wait()
    tokc = pltpu.make_async_copy(tokbuf, tokout_hbm.at[pl.ds(b0, L)], ts_sem)
    tokc.start()

  scatters = {}
  gathers = {}

  def start_gather(i):
    r = i % NBUF
    h = pltpu.make_async_copy(
        tbl_hbm.at[ridx_v.at[pl.ds(i * ROW_STRIDE, ROW_STRIDE)]],
        stages[r], gsems[r])
    h.start()
    gathers[i] = (h,)

  start_gather(0)
  start_gather(1)
  for i in range(BPW):
    r = i % NBUF
    if i + 2 < BPW:
      if i + 2 >= NBUF:
        for sh in scatters.pop(i + 2 - NBUF):
          sh.wait()
      start_gather(i + 2)
    for h in gathers.pop(i):
      h.wait()
    sh0 = pltpu.make_async_copy(stages[r].at[pl.ds(0, 72)],
                                out_hbm.at[base + i, pl.ds(0, 72)], ssems[r])
    sh1 = pltpu.make_async_copy(stages[r].at[pl.ds(72, 5)],
                                out_hbm.at[base + i, pl.ds(72, 5)], ssems[r])
    sh0.start()
    sh1.start()
    scatters[i] = (sh0, sh1)

  for i in sorted(scatters):
    for sh in scatters[i]:
      sh.wait()
  tokc.wait()


@jax.jit
def _prompt_gather(label, ctx, token_prefix, token_suffix, tokenized_prompts):
  tbl = jnp.concatenate([token_prefix, ctx, token_suffix],
                        axis=1).reshape(N_CLS * SEQ_LEN, CTX_DIM)
  tok_r = jnp.pad(tokenized_prompts, ((0, 0), (0, TOK_PAD - SEQ_LEN)))

  mesh = plsc.VectorSubcoreMesh(core_axis_name="c", subcore_axis_name="s")
  run = pl.kernel(
      _sc_gather_body,
      out_type=(
          jax.ShapeDtypeStruct((BATCH, SEQ_LEN, CTX_DIM), jnp.float32),
          jax.ShapeDtypeStruct((BATCH, TOK_PAD), jnp.int32),
      ),
      mesh=mesh,
      scratch_types=[
          pltpu.VMEM((BPW,), jnp.int32),
          pltpu.VMEM((BPW * ROW_STRIDE,), jnp.int32),
          pltpu.VMEM((ROW_STRIDE, CTX_DIM), jnp.float32),
          pltpu.VMEM((ROW_STRIDE, CTX_DIM), jnp.float32),
          pltpu.VMEM((ROW_STRIDE, CTX_DIM), jnp.float32),
          pltpu.VMEM((L, TOK_PAD), jnp.int32),
          [pltpu.SemaphoreType.DMA] * NBUF,
          [pltpu.SemaphoreType.DMA] * NBUF,
          pltpu.SemaphoreType.DMA,
          pltpu.SemaphoreType.DMA,
      ],
      compiler_params=pltpu.CompilerParams(needs_layout_passes=False),
  )
  prompts, tok_padded = run(label, tbl, tok_r)
  return prompts, tok_padded[:, :SEQ_LEN]


def kernel(label, ctx, token_prefix, token_suffix, tokenized_prompts):
  return _prompt_gather(label, ctx, token_prefix, token_suffix,
                        tokenized_prompts)


# speedup vs baseline: 1.0799x; 1.0044x over previous
import jax
import jax.numpy as jnp
from jax import lax
from jax.experimental import pallas as pl
from jax.experimental.pallas import tpu as pltpu
from jax.experimental.pallas import tpu_sc as plsc

N_CLS = 100
N_CTX = 16
CTX_DIM = 512
SEQ_LEN = 77
SUFFIX_LEN = SEQ_LEN - 1 - N_CTX
BATCH = 1024

NC, NS, L = 2, 16, 16
NW = NC * NS
BPW = BATCH // NW

TOK_PAD = 128
ROW_STRIDE = 80
NBUF = 3


def _sc_gather_body(label_hbm, tbl_hbm, tok_hbm, out_hbm, tokout_hbm,
                    idx_v, ridx_v, stage0, stage1, stage2, tokbuf,
                    gsems, ssems, tg_sem, ts_sem):
  wid = lax.axis_index("s") * NC + lax.axis_index("c")
  base = wid * BPW
  stages = (stage0, stage1, stage2)

  pltpu.sync_copy(label_hbm.at[pl.ds(base, BPW)], idx_v)

  iota = lax.iota(jnp.int32, L)

  for g in range(BPW // L):
    lab = idx_v[pl.ds(g * L, L)]
    pos = iota * ROW_STRIDE + g * L * ROW_STRIDE
    for k in range(ROW_STRIDE):
      plsc.store_scatter(ridx_v, [pos + k],
                         lab * SEQ_LEN + min(k, SEQ_LEN - 1))

  scatters = {}
  gathers = {}

  def start_gather(i):
    r = i % NBUF
    h = pltpu.make_async_copy(
        tbl_hbm.at[ridx_v.at[pl.ds(i * ROW_STRIDE, ROW_STRIDE)]],
        stages[r], gsems[r])
    h.start()
    gathers[i] = (h,)

  start_gather(0)
  start_gather(1)

  tokc = None
  for g in range(BPW // L):
    b0 = base + g * L
    if g > 0:
      tokc.wait()
    tg = pltpu.make_async_copy(tok_hbm.at[idx_v.at[pl.ds(g * L, L)]],
                               tokbuf, tg_sem)
    tg.start()
    tg.wait()
    tokc = pltpu.make_async_copy(tokbuf, tokout_hbm.at[pl.ds(b0, L)], ts_sem)
    tokc.start()

  for i in range(BPW):
    r = i % NBUF
    if i + 2 < BPW:
      if i + 2 >= NBUF:
        for sh in scatters.pop(i + 2 - NBUF):
          sh.wait()
      start_gather(i + 2)
    for h in gathers.pop(i):
      h.wait()
    sh0 = pltpu.make_async_copy(stages[r].at[pl.ds(0, 72)],
                                out_hbm.at[base + i, pl.ds(0, 72)], ssems[r])
    sh1 = pltpu.make_async_copy(stages[r].at[pl.ds(72, 5)],
                                out_hbm.at[base + i, pl.ds(72, 5)], ssems[r])
    sh0.start()
    sh1.start()
    scatters[i] = (sh0, sh1)

  for i in sorted(scatters):
    for sh in scatters[i]:
      sh.wait()
  tokc.wait()


@jax.jit
def _prompt_gather(label, ctx, token_prefix, token_suffix, tokenized_prompts):
  tbl = jnp.concatenate([token_prefix, ctx, token_suffix],
                        axis=1).reshape(N_CLS * SEQ_LEN, CTX_DIM)
  tok_r = jnp.pad(tokenized_prompts, ((0, 0), (0, TOK_PAD - SEQ_LEN)))

  mesh = plsc.VectorSubcoreMesh(core_axis_name="c", subcore_axis_name="s")
  run = pl.kernel(
      _sc_gather_body,
      out_type=(
          jax.ShapeDtypeStruct((BATCH, SEQ_LEN, CTX_DIM), jnp.float32),
          jax.ShapeDtypeStruct((BATCH, TOK_PAD), jnp.int32),
      ),
      mesh=mesh,
      scratch_types=[
          pltpu.VMEM((BPW,), jnp.int32),
          pltpu.VMEM((BPW * ROW_STRIDE,), jnp.int32),
          pltpu.VMEM((ROW_STRIDE, CTX_DIM), jnp.float32),
          pltpu.VMEM((ROW_STRIDE, CTX_DIM), jnp.float32),
          pltpu.VMEM((ROW_STRIDE, CTX_DIM), jnp.float32),
          pltpu.VMEM((L, TOK_PAD), jnp.int32),
          [pltpu.SemaphoreType.DMA] * NBUF,
          [pltpu.SemaphoreType.DMA] * NBUF,
          pltpu.SemaphoreType.DMA,
          pltpu.SemaphoreType.DMA,
      ],
      compiler_params=pltpu.CompilerParams(needs_layout_passes=False),
  )
  prompts, tok_padded = run(label, tbl, tok_r)
  return prompts, tok_padded[:, :SEQ_LEN]


def kernel(label, ctx, token_prefix, token_suffix, tokenized_prompts):
  return _prompt_gather(label, ctx, token_prefix, token_suffix,
                        tokenized_prompts)
